# baseline (speedup 1.0000x reference)
import functools

import jax
import jax.numpy as jnp
from jax import lax
from jax.experimental import pallas as pl
from jax.experimental.pallas import tpu as pltpu
from jax.experimental.pallas import tpu_sc as plsc

N_USERS = 100000
N_ENTITIES = 100000
N_RELATIONS = 32
DIM = 32
B = 4096
N_NEIGHBOR = 16
HALF = 16

NW = 32
CHUNK = B // NW
E = 4
NB = CHUNK // E
R2 = E * 16
NC2 = 2 * E


_GDN = lax.GatherDimensionNumbers(
    offset_dims=(), collapsed_slice_dims=(0,), start_index_map=(0,))


def _vgather16(v, idx):
    return lax.gather(v, idx[:, None], _GDN, (1,),
                      mode=lax.GatherScatterMode.PROMISE_IN_BOUNDS)


def _srel_lookup(sr0, sr1, r):
    rl = jnp.bitwise_and(r, HALF - 1)
    lo = _vgather16(sr0, rl)
    hi = _vgather16(sr1, rl)
    return jnp.where(r < HALF, lo, hi)


def _softmax_div16(s):
    iota = lax.iota(jnp.int32, HALF)
    m = s
    for step in (8, 4, 2, 1):
        m = jnp.maximum(m, _vgather16(m, iota ^ step))
    ex = jnp.exp(s - m)
    ssum = ex
    for step in (8, 4, 2, 1):
        ssum = ssum + _vgather16(ssum, iota ^ step)
    return ex * ((1.0 / N_NEIGHBOR) / ssum)


ROW_U = N_ENTITIES
ROW_E = N_ENTITIES + N_USERS


def _f32(v):
    return lax.bitcast_convert_type(v, jnp.float32)


def _sc_kernel(i_ids_hbm, u_ids_hbm, tab_hbm, relt_hbm,
               x0_out, x1_out, wn0_out,
               relt_v, iidx_v, iidx2_v, uidx_v, adjer_v, e1flat_v, e1flat2_v,
               adjer2_v, e2flat_v, u_v, e0_v, e1rows_v, e2rows_v,
               wn0_v, agg0_v, agg1_v,
               sem_l1, sem_ei0, sem_ei1, sem_l20, sem_l21, sem_e20, sem_e21):
    wid = lax.axis_index("s") * 2 + lax.axis_index("c")
    cbase = wid * CHUNK
    sem_ei = [sem_ei0, sem_ei1]
    sem_l2 = [sem_l20, sem_l21]
    sem_e2 = [sem_e20, sem_e21]

    def fire_level2(bb, p):
        pltpu.async_copy(tab_hbm.at[e1flat_v.at[pl.ds(bb * R2, R2)]],
                         adjer2_v.at[p], sem_ei[p])
        pltpu.async_copy(tab_hbm.at[e1flat2_v.at[pl.ds(bb * R2, R2)]],
                         e1rows_v.at[p], sem_l2[p])

    def wait_e2idx(p):
        pltpu.make_async_copy(tab_hbm.at[e1flat_v.at[pl.ds(0, R2)]],
                              adjer2_v.at[p], sem_ei[p]).wait()

    def drain_level2(p):
        pltpu.make_async_copy(tab_hbm.at[e1flat2_v.at[pl.ds(0, R2)]],
                              e1rows_v.at[p], sem_l2[p]).wait()

    def prep_e2rows(p):
        for r in range(R2):
            e2flat_v[p, r // 8, pl.ds((r % 8) * 16, 16)] = (
                adjer2_v[p, r, pl.ds(0, 16)] + ROW_E)
        for c in range(NC2):
            pltpu.async_copy(tab_hbm.at[e2flat_v.at[p, c]],
                             e2rows_v.at[p, c], sem_e2[p])

    def drain_e2rows(p):
        for c in range(NC2):
            pltpu.make_async_copy(tab_hbm.at[e2flat_v.at[p, c]],
                                  e2rows_v.at[p, c], sem_e2[p]).wait()

    def compute_block(bb, p):
        def elem_body(e, carry2):
            el = bb * E + e
            acc0 = jnp.zeros((HALF,), jnp.float32)
            acc1 = jnp.zeros((HALF,), jnp.float32)
            uh0 = _f32(u_v[el, pl.ds(0, HALF)])
            uh1 = _f32(u_v[el, pl.ds(HALF, HALF)])
            for d in range(DIM):
                us = uh0[d] if d < HALF else uh1[d - HALF]
                ud = jnp.broadcast_to(us, (HALF,))
                acc0 = acc0 + ud * relt_v[d, pl.ds(0, HALF)]
                acc1 = acc1 + ud * relt_v[d, pl.ds(HALF, HALF)]
            sr0 = acc0 * (1.0 / DIM)
            sr1 = acc1 * (1.0 / DIM)

            s0 = _srel_lookup(sr0, sr1, adjer_v[el, pl.ds(HALF, HALF)])
            wn = _softmax_div16(s0)
            wn0_v[el, :] = wn
            a0 = _f32(e0_v[el, pl.ds(0, HALF)])
            a1 = _f32(e0_v[el, pl.ds(HALF, HALF)])
            for j in range(N_NEIGHBOR):
                wj = jnp.broadcast_to(wn[j], (HALF,))
                a0 = a0 + wj * _f32(e1rows_v[p, e * 16 + j, pl.ds(0, HALF)])
                a1 = a1 + wj * _f32(e1rows_v[p, e * 16 + j,
                                             pl.ds(HALF, HALF)])
            agg0_v[el, pl.ds(0, HALF)] = a0
            agg0_v[el, pl.ds(HALF, HALF)] = a1

            def k_body(k, carry3):
                row = e * 16 + k
                s1 = _srel_lookup(sr0, sr1, adjer2_v[p, row, pl.ds(HALF, HALF)])
                wn1 = _softmax_div16(s1)
                c = e * 2 + k // 8
                rbase = (k % 8) * 16
                b0 = _f32(e1rows_v[p, row, pl.ds(0, HALF)])
                b1 = _f32(e1rows_v[p, row, pl.ds(HALF, HALF)])
                for j in range(N_NEIGHBOR):
                    wj = jnp.broadcast_to(wn1[j], (HALF,))
                    b0 = b0 + wj * _f32(e2rows_v[p, c, rbase + j,
                                                 pl.ds(0, HALF)])
                    b1 = b1 + wj * _f32(e2rows_v[p, c, rbase + j,
                                                 pl.ds(HALF, HALF)])
                agg1_v[row, pl.ds(0, HALF)] = b0
                agg1_v[row, pl.ds(HALF, HALF)] = b1
                return carry3

            lax.fori_loop(0, N_NEIGHBOR, k_body, 0)
            return carry2

        lax.fori_loop(0, E, elem_body, 0)

        pltpu.sync_copy(agg1_v,
                        x1_out.at[pl.ds((cbase + bb * E) * 16, R2)])

    pltpu.sync_copy(relt_hbm, relt_v)
    pltpu.sync_copy(i_ids_hbm.at[pl.ds(cbase, CHUNK)], iidx_v)
    pltpu.sync_copy(u_ids_hbm.at[pl.ds(cbase, CHUNK)], uidx_v)
    for g in range(CHUNK // HALF):
        iidx2_v[pl.ds(g * HALF, HALF)] = (
            iidx_v[pl.ds(g * HALF, HALF)] + ROW_E)
        uidx_v[pl.ds(g * HALF, HALF)] = (
            uidx_v[pl.ds(g * HALF, HALF)] + ROW_U)
    cps = [
        pltpu.async_copy(tab_hbm.at[iidx_v], adjer_v, sem_l1),
        pltpu.async_copy(tab_hbm.at[iidx2_v], e0_v, sem_l1),
        pltpu.async_copy(tab_hbm.at[uidx_v], u_v, sem_l1),
    ]
    for cp in cps:
        cp.wait()
    for r in range(CHUNK):
        e1 = adjer_v[r, pl.ds(0, HALF)]
        e1flat_v[pl.ds(r * 16, 16)] = e1
        e1flat2_v[pl.ds(r * 16, 16)] = e1 + ROW_E

    fire_level2(0, 0)
    fire_level2(1, 1)
    wait_e2idx(0)
    prep_e2rows(0)

    def t_body(t, carry):
        for ph in range(2):
            b = t * 2 + ph
            p, q = ph, 1 - ph
            drain_e2rows(p)
            drain_level2(p)
            wait_e2idx(q)
            prep_e2rows(q)
            compute_block(b, p)
            fire_level2(b + 2, p)
        return carry

    lax.fori_loop(0, NB // 2 - 1, t_body, 0)

    drain_e2rows(0)
    drain_level2(0)
    wait_e2idx(1)
    prep_e2rows(1)
    compute_block(NB - 2, 0)
    drain_e2rows(1)
    drain_level2(1)
    compute_block(NB - 1, 1)

    pltpu.sync_copy(agg0_v, x0_out.at[pl.ds(cbase, CHUNK)])
    pltpu.sync_copy(wn0_v, wn0_out.at[pl.ds(cbase, CHUNK)])


def _sc_gather_aggregate(i_ids, u_ids, tab, relt):
    f32 = jnp.float32
    kern = functools.partial(
        pl.kernel,
        out_type=[
            jax.ShapeDtypeStruct((B, DIM), f32),
            jax.ShapeDtypeStruct((B * 16, DIM), f32),
            jax.ShapeDtypeStruct((B, N_NEIGHBOR), f32),
        ],
        mesh=plsc.VectorSubcoreMesh(core_axis_name="c", subcore_axis_name="s"),
        compiler_params=pltpu.CompilerParams(use_tc_tiling_on_sc=False),
        scratch_types=[
            pltpu.VMEM((DIM, DIM), f32),
            pltpu.VMEM((CHUNK,), jnp.int32),
            pltpu.VMEM((CHUNK,), jnp.int32),
            pltpu.VMEM((CHUNK,), jnp.int32),
            pltpu.VMEM((CHUNK, 2 * 16), jnp.int32),
            pltpu.VMEM((CHUNK * 16,), jnp.int32),
            pltpu.VMEM((CHUNK * 16,), jnp.int32),
            pltpu.VMEM((2, R2, 2 * 16), jnp.int32),
            pltpu.VMEM((2, NC2, 128), jnp.int32),
            pltpu.VMEM((CHUNK, DIM), jnp.int32),
            pltpu.VMEM((CHUNK, DIM), jnp.int32),
            pltpu.VMEM((2, R2, DIM), jnp.int32),
            pltpu.VMEM((2, NC2, 128, DIM), jnp.int32),
            pltpu.VMEM((CHUNK, N_NEIGHBOR), f32),
            pltpu.VMEM((CHUNK, DIM), f32),
            pltpu.VMEM((R2, DIM), f32),
            pltpu.SemaphoreType.DMA,
            pltpu.SemaphoreType.DMA,
            pltpu.SemaphoreType.DMA,
            pltpu.SemaphoreType.DMA,
            pltpu.SemaphoreType.DMA,
            pltpu.SemaphoreType.DMA,
            pltpu.SemaphoreType.DMA,
        ],
    )(_sc_kernel)
    return kern(i_ids, u_ids, tab, relt)


BLK = 256


def _tc_kernel(x0_ref, x1_ref, wn_ref, w0t_ref, b0_ref,
               w1t_ref, b1_ref, out_ref):
    dot = lambda x, w: lax.dot_general(
        x, w, (((1,), (0,)), ((), ())),
        precision=lax.Precision.HIGHEST, preferred_element_type=jnp.float32)
    w0t = w0t_ref[...]
    b0 = b0_ref[...]
    h0 = jnp.maximum(dot(x0_ref[...], w0t) + b0, 0.0)
    h1 = jnp.maximum(dot(x1_ref[...], w0t) + b0, 0.0)
    h1r = h1.reshape(BLK, N_NEIGHBOR, DIM)
    aggp = jnp.sum(h1r * wn_ref[...][:, :, None], axis=1)
    out_ref[...] = jnp.tanh(dot(h0 + aggp, w1t_ref[...]) + b1_ref[...])


def _tc_epilogue(x0, x1, wn0, w0t, b0, w1t, b1):
    grid = (B // BLK,)
    return pl.pallas_call(
        _tc_kernel,
        grid=grid,
        in_specs=[
            pl.BlockSpec((BLK, DIM), lambda i: (i, 0)),
            pl.BlockSpec((BLK * 16, DIM), lambda i: (i, 0)),
            pl.BlockSpec((BLK, N_NEIGHBOR), lambda i: (i, 0)),
            pl.BlockSpec((DIM, DIM), lambda i: (0, 0)),
            pl.BlockSpec((1, DIM), lambda i: (0, 0)),
            pl.BlockSpec((DIM, DIM), lambda i: (0, 0)),
            pl.BlockSpec((1, DIM), lambda i: (0, 0)),
        ],
        out_specs=pl.BlockSpec((BLK, DIM), lambda i: (i, 0)),
        out_shape=jax.ShapeDtypeStruct((B, DIM), jnp.float32),
    )(x0, x1, wn0, w0t, b0, w1t, b1)


def kernel(data, adj_entity, adj_relation, user_emb, entity_emb, relation_emb,
           W0, b0, W1, b1):
    u_ids = data[:, 0].astype(jnp.int32)
    i_ids = data[:, 1].astype(jnp.int32)
    relt = relation_emb.T
    tab = jnp.concatenate([
        jnp.concatenate([adj_entity.astype(jnp.int32),
                         adj_relation.astype(jnp.int32)], axis=1),
        lax.bitcast_convert_type(user_emb, jnp.int32),
        lax.bitcast_convert_type(entity_emb, jnp.int32),
    ], axis=0)

    x0, x1, wn0 = _sc_gather_aggregate(i_ids, u_ids, tab, relt)

    return _tc_epilogue(x0, x1, wn0, W0.T, b0.reshape(1, DIM),
                        W1.T, b1.reshape(1, DIM))


# speedup vs baseline: 1.2975x; 1.2975x over previous
import functools

import jax
import jax.numpy as jnp
from jax import lax
from jax.experimental import pallas as pl
from jax.experimental.pallas import tpu as pltpu
from jax.experimental.pallas import tpu_sc as plsc

N_USERS = 100000
N_ENTITIES = 100000
N_RELATIONS = 32
DIM = 32
B = 4096
N_NEIGHBOR = 16
HALF = 16

NW = 32
CHUNK = B // NW
E = 4
NB = CHUNK // E
R2 = E * 16
NC2 = 2 * E


_GDN = lax.GatherDimensionNumbers(
    offset_dims=(), collapsed_slice_dims=(0,), start_index_map=(0,))


def _vgather16(v, idx):
    return lax.gather(v, idx[:, None], _GDN, (1,),
                      mode=lax.GatherScatterMode.PROMISE_IN_BOUNDS)


def _srel_lookup(sr0, sr1, r):
    rl = jnp.bitwise_and(r, HALF - 1)
    lo = _vgather16(sr0, rl)
    hi = _vgather16(sr1, rl)
    return jnp.where(r < HALF, lo, hi)


def _softmax_div16(s):
    iota = lax.iota(jnp.int32, HALF)
    m = s
    for step in (8, 4, 2, 1):
        m = jnp.maximum(m, _vgather16(m, iota ^ step))
    ex = jnp.exp(s - m)
    ssum = ex
    for step in (8, 4, 2, 1):
        ssum = ssum + _vgather16(ssum, iota ^ step)
    return ex * ((1.0 / N_NEIGHBOR) / ssum)


def _sc_kernel(i_ids_hbm, u_ids_hbm, adj2_hbm, emb2_hbm, relt_hbm,
               x0_out, x1_out, wn0_out,
               relt_v, iidx_v, iidx2_v, uidx_v, adjer_v, e1flat_v, e1flat2_v,
               adjer2_v, e2flat_v, u_v, e0_v, e1rows_v, e2rows_v,
               wn0_v, agg0_v, agg1_v,
               sem_l1, sem_ei0, sem_ei1, sem_l20, sem_l21, sem_e20, sem_e21):
    wid = lax.axis_index("s") * 2 + lax.axis_index("c")
    cbase = wid * CHUNK
    sem_ei = [sem_ei0, sem_ei1]
    sem_l2 = [sem_l20, sem_l21]
    sem_e2 = [sem_e20, sem_e21]

    def fire_level2(bb, p):
        pltpu.async_copy(adj2_hbm.at[e1flat_v.at[pl.ds(bb * R2, R2)]],
                         adjer2_v.at[p], sem_ei[p])
        pltpu.async_copy(emb2_hbm.at[e1flat2_v.at[pl.ds(bb * R2, R2)]],
                         e1rows_v.at[p], sem_l2[p])

    def wait_e2idx(p):
        pltpu.make_async_copy(adj2_hbm.at[e1flat_v.at[pl.ds(0, R2)]],
                              adjer2_v.at[p], sem_ei[p]).wait()

    def drain_level2(p):
        pltpu.make_async_copy(emb2_hbm.at[e1flat2_v.at[pl.ds(0, R2)]],
                              e1rows_v.at[p], sem_l2[p]).wait()

    def prep_e2rows(p):
        for r in range(R2):
            e2flat_v[p, r // 8, pl.ds((r % 8) * 16, 16)] = (
                adjer2_v[p, r, pl.ds(0, 16)] + N_USERS)
        for c in range(NC2):
            pltpu.async_copy(emb2_hbm.at[e2flat_v.at[p, c]],
                             e2rows_v.at[p, c], sem_e2[p])

    def drain_e2rows(p):
        for c in range(NC2):
            pltpu.make_async_copy(emb2_hbm.at[e2flat_v.at[p, c]],
                                  e2rows_v.at[p, c], sem_e2[p]).wait()

    def compute_block(bb, p):
        def elem_body(e, carry2):
            el = bb * E + e
            acc0 = jnp.zeros((HALF,), jnp.float32)
            acc1 = jnp.zeros((HALF,), jnp.float32)
            uh0 = u_v[el, pl.ds(0, HALF)]
            uh1 = u_v[el, pl.ds(HALF, HALF)]
            for d in range(DIM):
                us = uh0[d] if d < HALF else uh1[d - HALF]
                ud = jnp.broadcast_to(us, (HALF,))
                acc0 = acc0 + ud * relt_v[d, pl.ds(0, HALF)]
                acc1 = acc1 + ud * relt_v[d, pl.ds(HALF, HALF)]
            sr0 = acc0 * (1.0 / DIM)
            sr1 = acc1 * (1.0 / DIM)

            s0 = _srel_lookup(sr0, sr1, adjer_v[el, pl.ds(HALF, HALF)])
            wn = _softmax_div16(s0)
            wn0_v[el, :] = wn
            a0 = e0_v[el, pl.ds(0, HALF)]
            a1 = e0_v[el, pl.ds(HALF, HALF)]
            for j in range(N_NEIGHBOR):
                wj = jnp.broadcast_to(wn[j], (HALF,))
                a0 = a0 + wj * e1rows_v[p, e * 16 + j, pl.ds(0, HALF)]
                a1 = a1 + wj * e1rows_v[p, e * 16 + j, pl.ds(HALF, HALF)]
            agg0_v[el, pl.ds(0, HALF)] = a0
            agg0_v[el, pl.ds(HALF, HALF)] = a1

            def k_body(k, carry3):
                row = e * 16 + k
                s1 = _srel_lookup(sr0, sr1, adjer2_v[p, row, pl.ds(HALF, HALF)])
                wn1 = _softmax_div16(s1)
                c = e * 2 + k // 8
                rbase = (k % 8) * 16
                b0 = e1rows_v[p, row, pl.ds(0, HALF)]
                b1 = e1rows_v[p, row, pl.ds(HALF, HALF)]
                for j in range(N_NEIGHBOR):
                    wj = jnp.broadcast_to(wn1[j], (HALF,))
                    b0 = b0 + wj * e2rows_v[p, c, rbase + j, pl.ds(0, HALF)]
                    b1 = b1 + wj * e2rows_v[p, c, rbase + j,
                                            pl.ds(HALF, HALF)]
                agg1_v[row, pl.ds(0, HALF)] = b0
                agg1_v[row, pl.ds(HALF, HALF)] = b1
                return carry3

            lax.fori_loop(0, N_NEIGHBOR, k_body, 0)
            return carry2

        lax.fori_loop(0, E, elem_body, 0)

        pltpu.sync_copy(agg1_v,
                        x1_out.at[pl.ds((cbase + bb * E) * 16, R2)])

    pltpu.sync_copy(relt_hbm, relt_v)
    pltpu.sync_copy(i_ids_hbm.at[pl.ds(cbase, CHUNK)], iidx_v)
    pltpu.sync_copy(u_ids_hbm.at[pl.ds(cbase, CHUNK)], uidx_v)
    for g in range(CHUNK // HALF):
        iidx2_v[pl.ds(g * HALF, HALF)] = (
            iidx_v[pl.ds(g * HALF, HALF)] + N_USERS)
    cps = [
        pltpu.async_copy(adj2_hbm.at[iidx_v], adjer_v, sem_l1),
        pltpu.async_copy(emb2_hbm.at[iidx2_v], e0_v, sem_l1),
        pltpu.async_copy(emb2_hbm.at[uidx_v], u_v, sem_l1),
    ]
    for cp in cps:
        cp.wait()
    for r in range(CHUNK):
        e1 = adjer_v[r, pl.ds(0, HALF)]
        e1flat_v[pl.ds(r * 16, 16)] = e1
        e1flat2_v[pl.ds(r * 16, 16)] = e1 + N_USERS

    fire_level2(0, 0)
    fire_level2(1, 1)
    wait_e2idx(0)
    prep_e2rows(0)

    def t_body(t, carry):
        for ph in range(2):
            b = t * 2 + ph
            p, q = ph, 1 - ph
            drain_e2rows(p)
            drain_level2(p)
            wait_e2idx(q)
            prep_e2rows(q)
            compute_block(b, p)
            fire_level2(b + 2, p)
        return carry

    lax.fori_loop(0, NB // 2 - 1, t_body, 0)

    drain_e2rows(0)
    drain_level2(0)
    wait_e2idx(1)
    prep_e2rows(1)
    compute_block(NB - 2, 0)
    drain_e2rows(1)
    drain_level2(1)
    compute_block(NB - 1, 1)

    pltpu.sync_copy(agg0_v, x0_out.at[pl.ds(cbase, CHUNK)])
    pltpu.sync_copy(wn0_v, wn0_out.at[pl.ds(cbase, CHUNK)])


def _sc_gather_aggregate(i_ids, u_ids, adj2, emb2, relt):
    f32 = jnp.float32
    kern = functools.partial(
        pl.kernel,
        out_type=[
            jax.ShapeDtypeStruct((B, DIM), f32),
            jax.ShapeDtypeStruct((B * 16, DIM), f32),
            jax.ShapeDtypeStruct((B, N_NEIGHBOR), f32),
        ],
        mesh=plsc.VectorSubcoreMesh(core_axis_name="c", subcore_axis_name="s"),
        compiler_params=pltpu.CompilerParams(use_tc_tiling_on_sc=False),
        scratch_types=[
            pltpu.VMEM((DIM, DIM), f32),
            pltpu.VMEM((CHUNK,), jnp.int32),
            pltpu.VMEM((CHUNK,), jnp.int32),
            pltpu.VMEM((CHUNK,), jnp.int32),
            pltpu.VMEM((CHUNK, 2 * 16), jnp.int32),
            pltpu.VMEM((CHUNK * 16,), jnp.int32),
            pltpu.VMEM((CHUNK * 16,), jnp.int32),
            pltpu.VMEM((2, R2, 2 * 16), jnp.int32),
            pltpu.VMEM((2, NC2, 128), jnp.int32),
            pltpu.VMEM((CHUNK, DIM), f32),
            pltpu.VMEM((CHUNK, DIM), f32),
            pltpu.VMEM((2, R2, DIM), f32),
            pltpu.VMEM((2, NC2, 128, DIM), f32),
            pltpu.VMEM((CHUNK, N_NEIGHBOR), f32),
            pltpu.VMEM((CHUNK, DIM), f32),
            pltpu.VMEM((R2, DIM), f32),
            pltpu.SemaphoreType.DMA,
            pltpu.SemaphoreType.DMA,
            pltpu.SemaphoreType.DMA,
            pltpu.SemaphoreType.DMA,
            pltpu.SemaphoreType.DMA,
            pltpu.SemaphoreType.DMA,
            pltpu.SemaphoreType.DMA,
        ],
    )(_sc_kernel)
    return kern(i_ids, u_ids, adj2, emb2, relt)


BLK = 256


def _tc_kernel(x0_ref, x1_ref, wn_ref, w0t_ref, b0_ref,
               w1t_ref, b1_ref, out_ref):
    dot = lambda x, w: lax.dot_general(
        x, w, (((1,), (0,)), ((), ())),
        precision=lax.Precision.HIGHEST, preferred_element_type=jnp.float32)
    w0t = w0t_ref[...]
    b0 = b0_ref[...]
    h0 = jnp.maximum(dot(x0_ref[...], w0t) + b0, 0.0)
    h1 = jnp.maximum(dot(x1_ref[...], w0t) + b0, 0.0)
    h1r = h1.reshape(BLK, N_NEIGHBOR, DIM)
    aggp = jnp.sum(h1r * wn_ref[...][:, :, None], axis=1)
    out_ref[...] = jnp.tanh(dot(h0 + aggp, w1t_ref[...]) + b1_ref[...])


def _tc_epilogue(x0, x1, wn0, w0t, b0, w1t, b1):
    grid = (B // BLK,)
    return pl.pallas_call(
        _tc_kernel,
        grid=grid,
        in_specs=[
            pl.BlockSpec((BLK, DIM), lambda i: (i, 0)),
            pl.BlockSpec((BLK * 16, DIM), lambda i: (i, 0)),
            pl.BlockSpec((BLK, N_NEIGHBOR), lambda i: (i, 0)),
            pl.BlockSpec((DIM, DIM), lambda i: (0, 0)),
            pl.BlockSpec((1, DIM), lambda i: (0, 0)),
            pl.BlockSpec((DIM, DIM), lambda i: (0, 0)),
            pl.BlockSpec((1, DIM), lambda i: (0, 0)),
        ],
        out_specs=pl.BlockSpec((BLK, DIM), lambda i: (i, 0)),
        out_shape=jax.ShapeDtypeStruct((B, DIM), jnp.float32),
    )(x0, x1, wn0, w0t, b0, w1t, b1)


def kernel(data, adj_entity, adj_relation, user_emb, entity_emb, relation_emb,
           W0, b0, W1, b1):
    u_ids = data[:, 0].astype(jnp.int32)
    i_ids = data[:, 1].astype(jnp.int32)
    relt = relation_emb.T
    adj2 = jnp.concatenate([adj_entity.astype(jnp.int32),
                            adj_relation.astype(jnp.int32)], axis=1)
    emb2 = jnp.concatenate([user_emb, entity_emb], axis=0)

    x0, x1, wn0 = _sc_gather_aggregate(i_ids, u_ids, adj2, emb2, relt)

    return _tc_epilogue(x0, x1, wn0, W0.T, b0.reshape(1, DIM),
                        W1.T, b1.reshape(1, DIM))
